# baseline (speedup 1.0000x reference)
import functools

import jax
import jax.numpy as jnp
from jax import lax
from jax.experimental import pallas as pl
from jax.experimental.pallas import tpu as pltpu
from jax.experimental.pallas import tpu_sc as plsc

INNER = 256
NCLS = 19
QLEN = 2975
BSZ = 8
HW = 128 * 128
NPX = BSZ * HW
TEMP = 0.2

NC, NS, L = 2, 16, 16
NW = NC * NS
PXT = NPX // NW
CHG = INNER // 4
ACC = NCLS * L * CHG
BLK = 128
NBLK = (QLEN + BLK - 1) // BLK
TAIL = QLEN - (NBLK - 1) * BLK


def _sc_mesh():
    return plsc.VectorSubcoreMesh(
        core_axis_name="c", subcore_axis_name="s", num_cores=NC, num_subcores=NS
    )


def _argmax_body(res_hbm, pred16_hbm, counts_hbm, resbuf, predbuf, cntbuf,
                 credbuf, sem):
    wid = lax.axis_index("s") * NC + lax.axis_index("c")
    b = wid // 4
    q = wid % 4
    base = b * (NCLS * HW) + q * PXT
    for c in range(NCLS):
        pltpu.async_copy(
            res_hbm.at[pl.ds(base + c * HW, PXT)],
            resbuf.at[pl.ds(c * PXT, PXT)],
            sem,
        )
    for c in range(NCLS):
        pltpu.make_async_copy(
            res_hbm.at[pl.ds(base + c * HW, PXT)],
            resbuf.at[pl.ds(c * PXT, PXT)],
            sem,
        ).wait()

    zero = jnp.zeros((L,), jnp.float32)
    for i in range(2 * L):
        cntbuf[pl.ds(i * L, L)] = zero

    lane = lax.iota(jnp.int32, L)
    lane64 = lane * CHG
    lane32 = lane * 32
    ones = jnp.ones((L,), jnp.float32)

    @pl.loop(0, PXT // L)
    def _px(v):
        off = v * L
        best = resbuf[pl.ds(off, L)]
        bidx = jnp.zeros((L,), jnp.int32)
        for c in range(1, NCLS):
            x = resbuf[pl.ds(c * PXT + off, L)]
            gt = x > best
            best = jnp.where(gt, x, best)
            bidx = jnp.where(gt, jnp.full((L,), c, jnp.int32), bidx)
        predbuf[pl.ds(off, L)] = bidx * (L * CHG) + lane64
        plsc.addupdate_scatter(cntbuf, [lane32 + bidx], ones)

    for kb in range(2):
        v = cntbuf[pl.ds(kb * L, L)]
        for l in range(1, L):
            v = v + cntbuf[pl.ds(l * 32 + kb * L, L)]
        credbuf[pl.ds(kb * L, L)] = v

    pltpu.sync_copy(predbuf, pred16_hbm.at[pl.ds(wid * PXT, PXT)])
    pltpu.sync_copy(credbuf, counts_hbm.at[wid])


@functools.cache
def _argmax_call():
    return pl.kernel(
        _argmax_body,
        out_type=[
            jax.ShapeDtypeStruct((NPX,), jnp.int32),
            jax.ShapeDtypeStruct((NW, 32), jnp.float32),
        ],
        mesh=_sc_mesh(),
        scratch_types=[
            pltpu.VMEM((NCLS * PXT,), jnp.float32),
            pltpu.VMEM((PXT,), jnp.int32),
            pltpu.VMEM((L * 32,), jnp.float32),
            pltpu.VMEM((32,), jnp.float32),
            pltpu.SemaphoreType.DMA,
        ],
        compiler_params=pltpu.CompilerParams(needs_layout_passes=False),
    )


NG = CHG // 8
NH = 4
PXQ = HW // NH
UN = NG * NH


def _segsum_body(fea_hbm, pred16_hbm, part_hbm, pbuf, fbuf, acc, racc,
                 sem0, sem1):
    wid = lax.axis_index("s") * NC + lax.axis_index("c")
    b = wid // 4
    cg = wid % 4

    pltpu.sync_copy(pred16_hbm.at[pl.ds(b * HW, HW)], pbuf)

    zero = jnp.zeros((L,), jnp.float32)

    @pl.loop(0, ACC // L)
    def _z(i):
        acc[pl.ds(i * L, L)] = zero

    febase = (b * INNER + cg * CHG) * HW
    sems = (sem0, sem1)

    def start_unit(u, slot):
        g = u // NH
        h = u % NH
        base = febase + g * (8 * HW) + h * PXQ
        for j in range(8):
            pltpu.async_copy(
                fea_hbm.at[pl.ds(base + j * HW, PXQ)],
                fbuf.at[pl.ds((slot * 8 + j) * PXQ, PXQ)],
                sems[slot],
            )

    def wait_unit(slot):
        for j in range(8):
            pltpu.make_async_copy(
                fea_hbm.at[pl.ds(0, PXQ)],
                fbuf.at[pl.ds((slot * 8 + j) * PXQ, PXQ)],
                sems[slot],
            ).wait()

    start_unit(0, 0)

    @pl.loop(0, UN, step=2)
    def _u(u0):
        for r in range(2):
            u = u0 + r

            @pl.when(u + 1 < UN)
            def _():
                start_unit(u + 1, 1 - r)

            wait_unit(r)
            g = u // NH
            h = u % NH
            koff = g * 8
            hoff = h * PXQ
            fbase = r * 8 * PXQ

            @plsc.parallel_loop(0, PXQ // L, unroll=8)
            def _px(v):
                off = v * L
                idx = pbuf[pl.ds(hoff + off, L)] + koff
                for j in range(8):
                    plsc.addupdate_scatter(
                        acc,
                        [idx if j == 0 else idx + j],
                        fbuf[pl.ds(fbase + j * PXQ + off, L)],
                    )

    @pl.loop(0, NCLS)
    def _red(c):
        cbase = c * (L * CHG)
        for kb in range(CHG // L):
            v = acc[pl.ds(cbase + kb * L, L)]
            for l in range(1, L):
                v = v + acc[pl.ds(cbase + l * CHG + kb * L, L)]
            racc[pl.ds(c * CHG + kb * L, L)] = v

    for c in range(NCLS):
        dst = part_hbm.at[pl.ds(((b * NCLS + c) * 4 + cg) * CHG, CHG)]
        pltpu.async_copy(racc.at[pl.ds(c * CHG, CHG)], dst, sem0)
    for c in range(NCLS):
        dst = part_hbm.at[pl.ds(((b * NCLS + c) * 4 + cg) * CHG, CHG)]
        pltpu.make_async_copy(racc.at[pl.ds(c * CHG, CHG)], dst, sem0).wait()


@functools.cache
def _segsum_call():
    return pl.kernel(
        _segsum_body,
        out_type=jax.ShapeDtypeStruct((BSZ * NCLS * 4 * CHG,), jnp.float32),
        mesh=_sc_mesh(),
        scratch_types=[
            pltpu.VMEM((HW,), jnp.int32),
            pltpu.VMEM((2 * 8 * PXQ,), jnp.float32),
            pltpu.VMEM((ACC,), jnp.float32),
            pltpu.VMEM((NCLS * CHG,), jnp.float32),
            pltpu.SemaphoreType.DMA,
            pltpu.SemaphoreType.DMA,
        ],
        compiler_params=pltpu.CompilerParams(needs_layout_passes=False),
    )


def _loss_body(pref, cref, qref, out, keys_b, l0s, sacc3):
    j = pl.program_id(0)
    invt = jnp.float32(1.0 / TEMP)
    qb = qref[...].reshape(NCLS, 4, CHG, BLK)

    @pl.when(j == 0)
    def _init():
        sums = jnp.sum(pref[...], axis=0)
        counts = jnp.sum(cref[...], axis=0)[:NCLS]
        safe = jnp.where(counts > 0, counts, jnp.ones_like(counts))
        k0 = sums / safe[:, None, None]
        nrm = jnp.sqrt(jnp.sum(k0 * k0, axis=(1, 2), keepdims=True))
        ks = k0 / jnp.maximum(nrm, 1e-12) * invt
        keys_b[...] = jnp.broadcast_to(
            ks[:, :, :, None], (NCLS, 4, CHG, BLK)
        )
        l0s[...] = ks * qb[:, :, :, 0]
        out[0, 0] = jnp.float32(0.0)

    qsum = jnp.sum(qb, axis=0)
    ks = keys_b[...]
    x1 = ks * qb
    x2 = ks * qsum[None] - x1
    e = jnp.exp(x1) + jnp.exp(x2)

    @pl.when(j == 0)
    def _acc0():
        sacc3[...] = e

    @pl.when(jnp.logical_and(j > 0, j < NBLK - 1))
    def _accmid():
        sacc3[...] = sacc3[...] + e

    @pl.when(j == NBLK - 1)
    def _fin():
        col = lax.broadcasted_iota(jnp.int32, (NCLS, 4, CHG, BLK), 3)
        em = jnp.where(col < TAIL, e, jnp.float32(0.0))
        s2 = jnp.sum(sacc3[...] + em, axis=3)
        counts = jnp.sum(cref[...], axis=0)[:NCLS]
        pres = (counts > 0).astype(jnp.float32)
        loss = jnp.sum(pres[:, None, None] * (jnp.log(s2) - l0s[...]))
        out[0, 0] = loss / jnp.float32(INNER)


def _loss_call(p4, c2, queues, interpret=False):
    return pl.pallas_call(
        _loss_body,
        grid=(NBLK,),
        in_specs=[
            pl.BlockSpec((BSZ, NCLS, 4, CHG), lambda j: (0, 0, 0, 0)),
            pl.BlockSpec((NW, 32), lambda j: (0, 0)),
            pl.BlockSpec((NCLS, INNER, BLK), lambda j: (0, 0, j)),
        ],
        out_specs=pl.BlockSpec(memory_space=pltpu.SMEM),
        out_shape=jax.ShapeDtypeStruct((1, 1), jnp.float32),
        scratch_shapes=[
            pltpu.VMEM((NCLS, 4, CHG, BLK), jnp.float32),
            pltpu.VMEM((NCLS, 4, CHG), jnp.float32),
            pltpu.VMEM((NCLS, 4, CHG, BLK), jnp.float32),
        ],
        compiler_params=pltpu.CompilerParams(
            dimension_semantics=("arbitrary",)
        ),
        interpret=interpret,
    )(p4, c2, queues)


def kernel(fea, res, queues):
    res_flat = res.reshape(-1)
    fea_flat = fea.reshape(-1)
    pred16, countsp = _argmax_call()(res_flat)
    partials = _segsum_call()(fea_flat, pred16)
    p4 = partials.reshape(BSZ, NCLS, 4, CHG)
    loss = _loss_call(p4, countsp, queues)[0, 0]
    return res, loss


# speedup vs baseline: 3.7941x; 3.7941x over previous
import functools

import jax
import jax.numpy as jnp
from jax import lax
from jax.experimental import pallas as pl
from jax.experimental.pallas import tpu as pltpu
from jax.experimental.pallas import tpu_sc as plsc

INNER = 256
NCLS = 19
QLEN = 2975
BSZ = 8
HW = 128 * 128
NPX = BSZ * HW
TEMP = 0.2

NC, NS, L = 2, 16, 16
NW = NC * NS
PXT = NPX // NW
CHG = INNER // 4
ACC = NCLS * L * CHG
BLK = 128
NBLK = (QLEN + BLK - 1) // BLK
TAIL = QLEN - (NBLK - 1) * BLK


def _sc_mesh():
    return plsc.VectorSubcoreMesh(
        core_axis_name="c", subcore_axis_name="s", num_cores=NC, num_subcores=NS
    )


def _argmax_body(res_hbm, pred16_hbm, counts_hbm, resbuf, predbuf, cntbuf,
                 credbuf, sem):
    wid = lax.axis_index("s") * NC + lax.axis_index("c")
    b = wid // 4
    q = wid % 4
    base = b * (NCLS * HW) + q * PXT
    for c in range(NCLS):
        pltpu.async_copy(
            res_hbm.at[pl.ds(base + c * HW, PXT)],
            resbuf.at[pl.ds(c * PXT, PXT)],
            sem,
        )
    for c in range(NCLS):
        pltpu.make_async_copy(
            res_hbm.at[pl.ds(base + c * HW, PXT)],
            resbuf.at[pl.ds(c * PXT, PXT)],
            sem,
        ).wait()

    zero = jnp.zeros((L,), jnp.float32)
    for i in range(NCLS):
        cntbuf[pl.ds(i * L, L)] = zero
    credbuf[pl.ds(0, L)] = zero
    credbuf[pl.ds(L, L)] = zero

    lane = lax.iota(jnp.int32, L)
    ones = jnp.ones((L,), jnp.float32)

    @pl.loop(0, PXT // L)
    def _px(v):
        off = v * L
        best = resbuf[pl.ds(off, L)]
        bidx = jnp.zeros((L,), jnp.int32)
        for c in range(1, NCLS):
            x = resbuf[pl.ds(c * PXT + off, L)]
            gt = x > best
            best = jnp.where(gt, x, best)
            bidx = jnp.where(gt, jnp.full((L,), c, jnp.int32), bidx)
        predbuf[pl.ds(off, L)] = bidx * (L * CHG) + lane
        plsc.addupdate_scatter(cntbuf, [bidx * L + lane], ones)

    m15 = lane == (L - 1)
    for c in range(NCLS):
        cs = plsc.cumsum(cntbuf[pl.ds(c * L, L)])
        plsc.store_scatter(
            credbuf, [jnp.full((L,), c, jnp.int32)], cs, mask=m15
        )

    pltpu.sync_copy(predbuf, pred16_hbm.at[pl.ds(wid * PXT, PXT)])
    pltpu.sync_copy(credbuf, counts_hbm.at[wid])


@functools.cache
def _argmax_call():
    return pl.kernel(
        _argmax_body,
        out_type=[
            jax.ShapeDtypeStruct((NPX,), jnp.int32),
            jax.ShapeDtypeStruct((NW, 32), jnp.float32),
        ],
        mesh=_sc_mesh(),
        scratch_types=[
            pltpu.VMEM((NCLS * PXT,), jnp.float32),
            pltpu.VMEM((PXT,), jnp.int32),
            pltpu.VMEM((NCLS * L,), jnp.float32),
            pltpu.VMEM((32,), jnp.float32),
            pltpu.SemaphoreType.DMA,
        ],
        compiler_params=pltpu.CompilerParams(needs_layout_passes=False),
    )


NG = CHG // 8
NH = 4
PXQ = HW // NH
UN = NG * NH


def _segsum_body(fea_hbm, pred16_hbm, part_hbm, pbuf, fbuf, acc, racc,
                 sem0, sem1):
    wid = lax.axis_index("s") * NC + lax.axis_index("c")
    b = wid // 4
    cg = wid % 4

    pltpu.sync_copy(pred16_hbm.at[pl.ds(b * HW, HW)], pbuf)

    zero = jnp.zeros((L,), jnp.float32)

    @pl.loop(0, ACC // L)
    def _z(i):
        acc[pl.ds(i * L, L)] = zero

    febase = (b * INNER + cg * CHG) * HW
    sems = (sem0, sem1)

    def start_unit(u, slot):
        g = u // NH
        h = u % NH
        base = febase + g * (8 * HW) + h * PXQ
        for j in range(8):
            pltpu.async_copy(
                fea_hbm.at[pl.ds(base + j * HW, PXQ)],
                fbuf.at[pl.ds((slot * 8 + j) * PXQ, PXQ)],
                sems[slot],
            )

    def wait_unit(slot):
        for j in range(8):
            pltpu.make_async_copy(
                fea_hbm.at[pl.ds(0, PXQ)],
                fbuf.at[pl.ds((slot * 8 + j) * PXQ, PXQ)],
                sems[slot],
            ).wait()

    start_unit(0, 0)

    @pl.loop(0, UN, step=2)
    def _u(u0):
        for r in range(2):
            u = u0 + r

            @pl.when(u + 1 < UN)
            def _():
                start_unit(u + 1, 1 - r)

            wait_unit(r)
            g = u // NH
            h = u % NH
            koff = g * (8 * L)
            hoff = h * PXQ
            fbase = r * 8 * PXQ

            @plsc.parallel_loop(0, PXQ // L, unroll=8)
            def _px(v):
                off = v * L
                idx = pbuf[pl.ds(hoff + off, L)] + koff
                for j in range(8):
                    plsc.addupdate_scatter(
                        acc,
                        [idx if j == 0 else idx + j * L],
                        fbuf[pl.ds(fbase + j * PXQ + off, L)],
                    )

    lane = lax.iota(jnp.int32, L)
    m15 = lane == (L - 1)

    @plsc.parallel_loop(0, NCLS * CHG, unroll=8)
    def _red(i):
        cs = plsc.cumsum(acc[pl.ds(i * L, L)])
        plsc.store_scatter(
            racc, [jnp.full((L,), i, jnp.int32)], cs, mask=m15
        )

    for c in range(NCLS):
        dst = part_hbm.at[pl.ds(((b * NCLS + c) * 4 + cg) * CHG, CHG)]
        pltpu.async_copy(racc.at[pl.ds(c * CHG, CHG)], dst, sem0)
    for c in range(NCLS):
        dst = part_hbm.at[pl.ds(((b * NCLS + c) * 4 + cg) * CHG, CHG)]
        pltpu.make_async_copy(racc.at[pl.ds(c * CHG, CHG)], dst, sem0).wait()


@functools.cache
def _segsum_call():
    return pl.kernel(
        _segsum_body,
        out_type=jax.ShapeDtypeStruct((BSZ * NCLS * 4 * CHG,), jnp.float32),
        mesh=_sc_mesh(),
        scratch_types=[
            pltpu.VMEM((HW,), jnp.int32),
            pltpu.VMEM((2 * 8 * PXQ,), jnp.float32),
            pltpu.VMEM((ACC,), jnp.float32),
            pltpu.VMEM((NCLS * CHG,), jnp.float32),
            pltpu.SemaphoreType.DMA,
            pltpu.SemaphoreType.DMA,
        ],
        compiler_params=pltpu.CompilerParams(needs_layout_passes=False),
    )


def _loss_body(pref, cref, qref, out, keys_b, l0s, sacc3):
    j = pl.program_id(0)
    invt = jnp.float32(1.0 / TEMP)
    qb = qref[...].reshape(NCLS, 4, CHG, BLK)

    @pl.when(j == 0)
    def _init():
        sums = jnp.sum(pref[...], axis=0)
        counts = jnp.sum(cref[...], axis=0)[:NCLS]
        safe = jnp.where(counts > 0, counts, jnp.ones_like(counts))
        k0 = sums / safe[:, None, None]
        nrm = jnp.sqrt(jnp.sum(k0 * k0, axis=(1, 2), keepdims=True))
        ks = k0 / jnp.maximum(nrm, 1e-12) * invt
        keys_b[...] = jnp.broadcast_to(
            ks[:, :, :, None], (NCLS, 4, CHG, BLK)
        )
        l0s[...] = ks * qb[:, :, :, 0]
        out[0, 0] = jnp.float32(0.0)

    qsum = jnp.sum(qb, axis=0)
    ks = keys_b[...]
    x1 = ks * qb
    x2 = ks * qsum[None] - x1
    e = jnp.exp(x1) + jnp.exp(x2)

    @pl.when(j == 0)
    def _acc0():
        sacc3[...] = e

    @pl.when(jnp.logical_and(j > 0, j < NBLK - 1))
    def _accmid():
        sacc3[...] = sacc3[...] + e

    @pl.when(j == NBLK - 1)
    def _fin():
        col = lax.broadcasted_iota(jnp.int32, (NCLS, 4, CHG, BLK), 3)
        em = jnp.where(col < TAIL, e, jnp.float32(0.0))
        s2 = jnp.sum(sacc3[...] + em, axis=3)
        counts = jnp.sum(cref[...], axis=0)[:NCLS]
        pres = (counts > 0).astype(jnp.float32)
        loss = jnp.sum(pres[:, None, None] * (jnp.log(s2) - l0s[...]))
        out[0, 0] = loss / jnp.float32(INNER)


def _loss_call(p4, c2, queues, interpret=False):
    return pl.pallas_call(
        _loss_body,
        grid=(NBLK,),
        in_specs=[
            pl.BlockSpec((BSZ, NCLS, 4, CHG), lambda j: (0, 0, 0, 0)),
            pl.BlockSpec((NW, 32), lambda j: (0, 0)),
            pl.BlockSpec((NCLS, INNER, BLK), lambda j: (0, 0, j)),
        ],
        out_specs=pl.BlockSpec(memory_space=pltpu.SMEM),
        out_shape=jax.ShapeDtypeStruct((1, 1), jnp.float32),
        scratch_shapes=[
            pltpu.VMEM((NCLS, 4, CHG, BLK), jnp.float32),
            pltpu.VMEM((NCLS, 4, CHG), jnp.float32),
            pltpu.VMEM((NCLS, 4, CHG, BLK), jnp.float32),
        ],
        compiler_params=pltpu.CompilerParams(
            dimension_semantics=("arbitrary",)
        ),
        interpret=interpret,
    )(p4, c2, queues)


def kernel(fea, res, queues):
    res_flat = res.reshape(-1)
    fea_flat = fea.reshape(-1)
    pred16, countsp = _argmax_call()(res_flat)
    partials = _segsum_call()(fea_flat, pred16)
    p4 = partials.reshape(BSZ, NCLS, 4, CHG)
    loss = _loss_call(p4, countsp, queues)[0, 0]
    return res, loss
